# Initial kernel scaffold; baseline (speedup 1.0000x reference)
import functools

import jax
import jax.numpy as jnp
from jax import lax
from jax.experimental import pallas as pl
from jax.experimental.pallas import tpu as pltpu
from jax.experimental.pallas import tpu_sc as plsc

_N = 10000
_E = 320000
_D_IN = 128
_H = 128
_D_E = 16
_NG = 8
_EMB = 16
_NGRAPH = 64
_OUT = 8

_NC = 2
_NS = 16
_L = 16
_NW = _NC * _NS
_EPT = _E // _NW
_C = 80
_NCHUNK = _EPT // _C
_RPT = _N // _NS
_ZR = 125


def _sc_gine_body(h_hbm, src_hbm, dst_hbm, emat_hbm, out_hbm,
                  src_v, dst_v, hrows, erows, stage, agg_sh, sem):
    cid = lax.axis_index("c")
    sid = lax.axis_index("s")
    wid = sid * _NC + cid
    nvec = _H // _L

    zero = jnp.zeros((_L,), jnp.float32)

    def zrow(i, carry):
        for j in range(nvec):
            stage[i, pl.ds(j * _L, _L)] = zero
        return carry

    lax.fori_loop(0, _ZR, zrow, 0)
    row0 = sid * _RPT
    for m in range(_RPT // _ZR):
        pltpu.sync_copy(stage, agg_sh.at[pl.ds(row0 + m * _ZR, _ZR)])
    plsc.subcore_barrier()

    def chunk(k, carry):
        base = wid * _EPT + k * _C
        pltpu.sync_copy(src_hbm.at[pl.ds(base, _C)], src_v)
        pltpu.sync_copy(dst_hbm.at[pl.ds(base, _C)], dst_v)
        pltpu.async_copy(h_hbm.at[src_v], hrows, sem).wait()
        pltpu.sync_copy(emat_hbm.at[pl.ds(base, _C)], erows)

        def row(i, c2):
            for j in range(nvec):
                sl = pl.ds(j * _L, _L)
                erows[i, sl] = jnp.maximum(hrows[i, sl] + erows[i, sl], 0.0)
            return c2

        lax.fori_loop(0, _C, row, 0)
        pltpu.sync_copy(erows, agg_sh.at[dst_v], add=True)
        return carry

    lax.fori_loop(0, _NCHUNK, chunk, 0)
    plsc.subcore_barrier()

    for m in range(_RPT // _ZR):
        r = row0 + m * _ZR
        pltpu.sync_copy(agg_sh.at[pl.ds(r, _ZR)], stage)
        pltpu.sync_copy(stage, out_hbm.at[cid, pl.ds(r, _ZR)])


_sc_gine = pl.kernel(
    _sc_gine_body,
    out_type=jax.ShapeDtypeStruct((_NC, _N, _H), jnp.float32),
    mesh=plsc.VectorSubcoreMesh(core_axis_name="c", subcore_axis_name="s"),
    scratch_types=[
        pltpu.VMEM((_C,), jnp.int32),
        pltpu.VMEM((_C,), jnp.int32),
        pltpu.VMEM((_C, _H), jnp.float32),
        pltpu.VMEM((_C, _H), jnp.float32),
        pltpu.VMEM((_ZR, _H), jnp.float32),
        pltpu.VMEM_SHARED((_N, _H), jnp.float32),
        pltpu.SemaphoreType.DMA,
    ],
)


_BE = 3200


def _edge_mlp_body(ea_ref, w_ref, b_ref, o0_ref, o1_ref, o2_ref):
    y = jnp.dot(ea_ref[...], w_ref[...],
                preferred_element_type=jnp.float32) + b_ref[...]
    o0_ref[...] = y[:, 0:_H]
    o1_ref[...] = y[:, _H:2 * _H]
    o2_ref[...] = y[:, 2 * _H:3 * _H]


def _edge_mlp(ea, wcat, bcat):
    eshape = jax.ShapeDtypeStruct((_E, _H), jnp.float32)
    return pl.pallas_call(
        _edge_mlp_body,
        grid=(_E // _BE,),
        in_specs=[
            pl.BlockSpec((_BE, _D_E), lambda i: (i, 0)),
            pl.BlockSpec((_D_E, 3 * _H), lambda i: (0, 0)),
            pl.BlockSpec((1, 3 * _H), lambda i: (0, 0)),
        ],
        out_specs=[
            pl.BlockSpec((_BE, _H), lambda i: (i, 0)),
            pl.BlockSpec((_BE, _H), lambda i: (i, 0)),
            pl.BlockSpec((_BE, _H), lambda i: (i, 0)),
        ],
        out_shape=[eshape, eshape, eshape],
    )(ea, wcat, bcat)


def _encoder_body(x_ref, ef_ref, wx_ref, we_ref, b_ref, g_ref, bt_ref, o_ref):
    h = jnp.dot(x_ref[...], wx_ref[...], preferred_element_type=jnp.float32)
    h = h + jnp.dot(ef_ref[...], we_ref[...],
                    preferred_element_type=jnp.float32) + b_ref[...]
    h = jnp.maximum(h, 0.0)
    m = jnp.mean(h, axis=0, keepdims=True)
    v = jnp.mean((h - m) ** 2, axis=0, keepdims=True)
    o_ref[...] = g_ref[...] * (h - m) * lax.rsqrt(v + 1e-5) + bt_ref[...]


def _encoder(x, emb_full, wx, wemb, b2, g2, bt2):
    return pl.pallas_call(
        _encoder_body,
        out_shape=jax.ShapeDtypeStruct((_N, _H), jnp.float32),
    )(x, emb_full, wx, wemb, b2, g2, bt2)


def _node_mlp_body(with_bn, h_ref, ag_ref, wa_ref, ba_ref, wb_ref, bb_ref,
                   g_ref, bt_ref, o_ref):
    hs = h_ref[...] + ag_ref[0] + ag_ref[1]
    t = jnp.maximum(
        jnp.dot(hs, wa_ref[...], preferred_element_type=jnp.float32)
        + ba_ref[...], 0.0)
    t = jnp.dot(t, wb_ref[...], preferred_element_type=jnp.float32) + bb_ref[...]
    if with_bn:
        m = jnp.mean(t, axis=0, keepdims=True)
        v = jnp.mean((t - m) ** 2, axis=0, keepdims=True)
        t = g_ref[...] * (t - m) * lax.rsqrt(v + 1e-5) + bt_ref[...]
        t = jnp.maximum(t, 0.0)
    else:
        t = jnp.maximum(t, 0.0)
    o_ref[...] = t


def _node_mlp(with_bn, h, agg, wa, ba2, wb, bb2, g2, bt2):
    return pl.pallas_call(
        functools.partial(_node_mlp_body, with_bn),
        out_shape=jax.ShapeDtypeStruct((_N, _H), jnp.float32),
    )(h, agg, wa, ba2, wb, bb2, g2, bt2)


def _readout_body(h_ref, batch_ref, wl1_ref, bl1_ref, wl2_ref, bl2_ref, o_ref):
    gids = lax.broadcasted_iota(jnp.int32, (_N, _NGRAPH), 1)
    onehot = (batch_ref[...] == gids).astype(jnp.float32)
    pooled = lax.dot_general(onehot, h_ref[...], (((0,), (0,)), ((), ())),
                             preferred_element_type=jnp.float32)
    t = jnp.dot(jnp.maximum(pooled, 0.0), wl1_ref[...],
                preferred_element_type=jnp.float32) + bl1_ref[...]
    o_ref[...] = jnp.dot(t, wl2_ref[...],
                         preferred_element_type=jnp.float32) + bl2_ref[...]


def _readout(h, batch2d, wl1, bl1_2, wl2, bl2_2):
    return pl.pallas_call(
        _readout_body,
        out_shape=jax.ShapeDtypeStruct((_NGRAPH, _OUT), jnp.float32),
    )(h, batch2d, wl1, bl1_2, wl2, bl2_2)


def kernel(x, edge_index, edge_attr, batch, emb, W_enc, b_enc, g0, bt0,
           We0, be0, W0a, b0a, W0b, b0b,
           We1, be1, W1a, b1a, W1b, b1b,
           We2, be2, W2a, b2a, W2b, b2b,
           g_bn, b_bn, Wl1, bl1, Wl2, bl2):
    row2 = lambda v: v.reshape(1, -1)
    src = edge_index[0]
    dst = edge_index[1]

    wcat = jnp.concatenate([We0, We1, We2], axis=1)
    bcat = jnp.concatenate([be0, be1, be2]).reshape(1, 3 * _H)
    e0, e1, e2 = _edge_mlp(edge_attr, wcat, bcat)

    emb_full = jnp.concatenate(
        [emb, jnp.broadcast_to(emb[0:1], (_N - _NG, _EMB))], axis=0)
    h = _encoder(x, emb_full, W_enc[:_D_IN], W_enc[_D_IN:],
                 row2(b_enc), row2(g0), row2(bt0))

    agg = _sc_gine(h, src, dst, e0)
    h = _node_mlp(False, h, agg, W0a, row2(b0a), W0b, row2(b0b),
                  row2(g_bn), row2(b_bn))
    agg = _sc_gine(h, src, dst, e1)
    h = _node_mlp(True, h, agg, W1a, row2(b1a), W1b, row2(b1b),
                  row2(g_bn), row2(b_bn))
    agg = _sc_gine(h, src, dst, e2)
    h = _node_mlp(True, h, agg, W2a, row2(b2a), W2b, row2(b2b),
                  row2(g_bn), row2(b_bn))

    return _readout(h, batch.reshape(_N, 1), Wl1, row2(bl1), Wl2, row2(bl2))


# speedup vs baseline: 2.5150x; 2.5150x over previous
import functools

import jax
import jax.numpy as jnp
from jax import lax
from jax.experimental import pallas as pl
from jax.experimental.pallas import tpu as pltpu
from jax.experimental.pallas import tpu_sc as plsc

_N = 10000
_E = 320000
_D_IN = 128
_H = 128
_D_E = 16
_NG = 8
_EMB = 16
_NGRAPH = 64
_OUT = 8

_NC = 2
_NS = 16
_L = 16
_NW = _NC * _NS
_EPT = _E // _NW
_C = 80
_NCHUNK = _EPT // _C
_RC = 80
_NRCHUNK = _N // _RC


def _sc_gine_body(h_hbm, src_hbm, dst_hbm, emat_hbm, out_hbm,
                  src_v, dst_v, hrows, erows, agg_sh, sem):
    cid = lax.axis_index("c")
    sid = lax.axis_index("s")
    wid = sid * _NC + cid
    nvec = _H // _L
    ntrip = (_NRCHUNK - sid + _NS - 1) // _NS

    zero = jnp.zeros((_L,), jnp.float32)

    def zrow(i, carry):
        for j in range(nvec):
            erows[i, pl.ds(j * _L, _L)] = zero
        return carry

    lax.fori_loop(0, _C, zrow, 0)

    def zchunk(k, carry):
        pltpu.sync_copy(erows, agg_sh.at[pl.ds((sid + k * _NS) * _RC, _RC)])
        return carry

    lax.fori_loop(0, ntrip, zchunk, 0)
    plsc.subcore_barrier()

    def chunk(k, carry):
        base = wid * _EPT + k * _C
        pltpu.sync_copy(src_hbm.at[pl.ds(base, _C)], src_v)
        pltpu.sync_copy(dst_hbm.at[pl.ds(base, _C)], dst_v)
        pltpu.async_copy(h_hbm.at[src_v], hrows, sem).wait()
        pltpu.sync_copy(emat_hbm.at[pl.ds(base, _C)], erows)

        def row(i, c2):
            for j in range(nvec):
                sl = pl.ds(j * _L, _L)
                erows[i, sl] = jnp.maximum(hrows[i, sl] + erows[i, sl], 0.0)
            return c2

        lax.fori_loop(0, _C, row, 0)
        pltpu.sync_copy(erows, agg_sh.at[dst_v], add=True)
        return carry

    lax.fori_loop(0, _NCHUNK, chunk, 0)
    plsc.subcore_barrier()

    def dchunk(k, carry):
        r = (sid + k * _NS) * _RC
        pltpu.sync_copy(agg_sh.at[pl.ds(r, _RC)], erows)
        pltpu.sync_copy(erows, out_hbm.at[cid, pl.ds(r, _RC)])
        return carry

    lax.fori_loop(0, ntrip, dchunk, 0)


_sc_gine = pl.kernel(
    _sc_gine_body,
    out_type=jax.ShapeDtypeStruct((_NC, _N, _H), jnp.float32),
    mesh=plsc.VectorSubcoreMesh(core_axis_name="c", subcore_axis_name="s"),
    scratch_types=[
        pltpu.VMEM((_C,), jnp.int32),
        pltpu.VMEM((_C,), jnp.int32),
        pltpu.VMEM((_C, _H), jnp.float32),
        pltpu.VMEM((_C, _H), jnp.float32),
        pltpu.VMEM_SHARED((_N, _H), jnp.float32),
        pltpu.SemaphoreType.DMA,
    ],
)


_BE = 3200


def _edge_mlp_body(ea_ref, w_ref, b_ref, o0_ref, o1_ref, o2_ref):
    y = jnp.dot(ea_ref[...], w_ref[...],
                preferred_element_type=jnp.float32) + b_ref[...]
    o0_ref[...] = y[:, 0:_H]
    o1_ref[...] = y[:, _H:2 * _H]
    o2_ref[...] = y[:, 2 * _H:3 * _H]


def _edge_mlp(ea, wcat, bcat):
    eshape = jax.ShapeDtypeStruct((_E, _H), jnp.float32)
    return pl.pallas_call(
        _edge_mlp_body,
        grid=(_E // _BE,),
        in_specs=[
            pl.BlockSpec((_BE, _D_E), lambda i: (i, 0)),
            pl.BlockSpec((_D_E, 3 * _H), lambda i: (0, 0)),
            pl.BlockSpec((1, 3 * _H), lambda i: (0, 0)),
        ],
        out_specs=[
            pl.BlockSpec((_BE, _H), lambda i: (i, 0)),
            pl.BlockSpec((_BE, _H), lambda i: (i, 0)),
            pl.BlockSpec((_BE, _H), lambda i: (i, 0)),
        ],
        out_shape=[eshape, eshape, eshape],
    )(ea, wcat, bcat)


def _encoder_body(x_ref, ef_ref, wx_ref, we_ref, b_ref, g_ref, bt_ref, o_ref):
    h = jnp.dot(x_ref[...], wx_ref[...], preferred_element_type=jnp.float32)
    h = h + jnp.dot(ef_ref[...], we_ref[...],
                    preferred_element_type=jnp.float32) + b_ref[...]
    h = jnp.maximum(h, 0.0)
    m = jnp.mean(h, axis=0, keepdims=True)
    v = jnp.mean((h - m) ** 2, axis=0, keepdims=True)
    o_ref[...] = g_ref[...] * (h - m) * lax.rsqrt(v + 1e-5) + bt_ref[...]


def _encoder(x, emb_full, wx, wemb, b2, g2, bt2):
    return pl.pallas_call(
        _encoder_body,
        out_shape=jax.ShapeDtypeStruct((_N, _H), jnp.float32),
    )(x, emb_full, wx, wemb, b2, g2, bt2)


def _node_mlp_body(with_bn, h_ref, ag_ref, wa_ref, ba_ref, wb_ref, bb_ref,
                   g_ref, bt_ref, o_ref):
    hs = h_ref[...] + ag_ref[0] + ag_ref[1]
    t = jnp.maximum(
        jnp.dot(hs, wa_ref[...], preferred_element_type=jnp.float32)
        + ba_ref[...], 0.0)
    t = jnp.dot(t, wb_ref[...], preferred_element_type=jnp.float32) + bb_ref[...]
    if with_bn:
        m = jnp.mean(t, axis=0, keepdims=True)
        v = jnp.mean((t - m) ** 2, axis=0, keepdims=True)
        t = g_ref[...] * (t - m) * lax.rsqrt(v + 1e-5) + bt_ref[...]
        t = jnp.maximum(t, 0.0)
    else:
        t = jnp.maximum(t, 0.0)
    o_ref[...] = t


def _node_mlp(with_bn, h, agg, wa, ba2, wb, bb2, g2, bt2):
    return pl.pallas_call(
        functools.partial(_node_mlp_body, with_bn),
        out_shape=jax.ShapeDtypeStruct((_N, _H), jnp.float32),
    )(h, agg, wa, ba2, wb, bb2, g2, bt2)


def _readout_body(h_ref, batch_ref, wl1_ref, bl1_ref, wl2_ref, bl2_ref, o_ref):
    gids = lax.broadcasted_iota(jnp.int32, (_N, _NGRAPH), 1)
    onehot = (batch_ref[...] == gids).astype(jnp.float32)
    pooled = lax.dot_general(onehot, h_ref[...], (((0,), (0,)), ((), ())),
                             preferred_element_type=jnp.float32)
    t = jnp.dot(jnp.maximum(pooled, 0.0), wl1_ref[...],
                preferred_element_type=jnp.float32) + bl1_ref[...]
    o_ref[...] = jnp.dot(t, wl2_ref[...],
                         preferred_element_type=jnp.float32) + bl2_ref[...]


def _readout(h, batch2d, wl1, bl1_2, wl2, bl2_2):
    return pl.pallas_call(
        _readout_body,
        out_shape=jax.ShapeDtypeStruct((_NGRAPH, _OUT), jnp.float32),
    )(h, batch2d, wl1, bl1_2, wl2, bl2_2)


def kernel(x, edge_index, edge_attr, batch, emb, W_enc, b_enc, g0, bt0,
           We0, be0, W0a, b0a, W0b, b0b,
           We1, be1, W1a, b1a, W1b, b1b,
           We2, be2, W2a, b2a, W2b, b2b,
           g_bn, b_bn, Wl1, bl1, Wl2, bl2):
    row2 = lambda v: v.reshape(1, -1)
    src = edge_index[0]
    dst = edge_index[1]

    wcat = jnp.concatenate([We0, We1, We2], axis=1)
    bcat = jnp.concatenate([be0, be1, be2]).reshape(1, 3 * _H)
    e0, e1, e2 = _edge_mlp(edge_attr, wcat, bcat)

    emb_full = jnp.concatenate(
        [emb, jnp.broadcast_to(emb[0:1], (_N - _NG, _EMB))], axis=0)
    h = _encoder(x, emb_full, W_enc[:_D_IN], W_enc[_D_IN:],
                 row2(b_enc), row2(g0), row2(bt0))

    agg = _sc_gine(h, src, dst, e0)
    h = _node_mlp(False, h, agg, W0a, row2(b0a), W0b, row2(b0b),
                  row2(g_bn), row2(b_bn))
    agg = _sc_gine(h, src, dst, e1)
    h = _node_mlp(True, h, agg, W1a, row2(b1a), W1b, row2(b1b),
                  row2(g_bn), row2(b_bn))
    agg = _sc_gine(h, src, dst, e2)
    h = _node_mlp(True, h, agg, W2a, row2(b2a), W2b, row2(b2b),
                  row2(g_bn), row2(b_bn))

    return _readout(h, batch.reshape(_N, 1), Wl1, row2(bl1), Wl2, row2(bl2))


# speedup vs baseline: 2.6513x; 1.0542x over previous
import functools

import jax
import jax.numpy as jnp
from jax import lax
from jax.experimental import pallas as pl
from jax.experimental.pallas import tpu as pltpu
from jax.experimental.pallas import tpu_sc as plsc

_N = 10000
_E = 320000
_D_IN = 128
_H = 128
_D_E = 16
_NG = 8
_EMB = 16
_NGRAPH = 64
_OUT = 8

_NC = 2
_NS = 16
_L = 16
_NW = _NC * _NS
_EPT = _E // _NW
_C = 40
_NCHUNK = _EPT // _C
_RC = 40
_NRCHUNK = _N // _RC


def _sc_gine_body(h_hbm, src_hbm, dst_hbm, emat_hbm, out_hbm,
                  svs, dvs, hrs, ers, agg_sh, sis, sgs, ses):
    cid = lax.axis_index("c")
    sid = lax.axis_index("s")
    wid = sid * _NC + cid
    nvec = _H // _L
    ntrip = (_NRCHUNK - sid + _NS - 1) // _NS

    zero = jnp.zeros((_L,), jnp.float32)

    @pl.loop(0, _C)
    def _(i):
        for j in range(nvec):
            ers[0][i, pl.ds(j * _L, _L)] = zero

    @pl.loop(0, ntrip)
    def _(k):
        pltpu.sync_copy(ers[0], agg_sh.at[pl.ds((sid + k * _NS) * _RC, _RC)])

    plsc.subcore_barrier()

    ebase = wid * _EPT

    def issue_idx(c, q):
        b = ebase + c * _C
        pltpu.async_copy(src_hbm.at[pl.ds(b, _C)], svs[q], sis[q])
        pltpu.async_copy(dst_hbm.at[pl.ds(b, _C)], dvs[q], sis[q])

    def wait_idx(c, q):
        b = ebase + c * _C
        pltpu.make_async_copy(src_hbm.at[pl.ds(b, _C)], svs[q], sis[q]).wait()
        pltpu.make_async_copy(dst_hbm.at[pl.ds(b, _C)], dvs[q], sis[q]).wait()

    def issue_rows(c, q, b):
        pltpu.async_copy(h_hbm.at[svs[q]], hrs[b], sgs[b])
        pltpu.async_copy(emat_hbm.at[pl.ds(ebase + c * _C, _C)], ers[b],
                         ses[b])

    def wait_rows(c, q, b):
        pltpu.make_async_copy(h_hbm.at[svs[q]], hrs[b], sgs[b]).wait()
        pltpu.make_async_copy(emat_hbm.at[pl.ds(ebase + c * _C, _C)], ers[b],
                              ses[b]).wait()

    def compute(b):
        hr = hrs[b]
        er = ers[b]

        @pl.loop(0, _C, unroll=4)
        def _(i):
            for j in range(nvec):
                sl = pl.ds(j * _L, _L)
                er[i, sl] = jnp.maximum(hr[i, sl] + er[i, sl], 0.0)

    def step(c, q, b, steady):
        wait_rows(c, q, b)
        compute(b)
        pltpu.sync_copy(ers[b], agg_sh.at[dvs[q]], add=True)
        if steady:
            @pl.when(c + 4 < _NCHUNK)
            def _():
                issue_idx(c + 4, q)

            @pl.when(c + 2 < _NCHUNK)
            def _():
                wait_idx(c + 2, (q + 2) % 4)
                issue_rows(c + 2, (q + 2) % 4, b)

    for q in range(4):
        issue_idx(q, q)
    for q in range(2):
        wait_idx(q, q)
        issue_rows(q, q, q)

    @pl.loop(0, _NCHUNK // 4)
    def _(m):
        c = 4 * m
        for u in range(4):
            step(c + u, u, u % 2, True)

    for u in range(_NCHUNK % 4):
        step((_NCHUNK // 4) * 4 + u, u, u % 2, False)
    plsc.subcore_barrier()

    @pl.loop(0, ntrip)
    def _(k):
        r = (sid + k * _NS) * _RC
        pltpu.sync_copy(agg_sh.at[pl.ds(r, _RC)], ers[0])
        pltpu.sync_copy(ers[0], out_hbm.at[cid, pl.ds(r, _RC)])


_sc_gine = pl.kernel(
    _sc_gine_body,
    out_type=jax.ShapeDtypeStruct((_NC, _N, _H), jnp.float32),
    mesh=plsc.VectorSubcoreMesh(core_axis_name="c", subcore_axis_name="s"),
    scratch_types=[
        [pltpu.VMEM((_C,), jnp.int32) for _ in range(4)],
        [pltpu.VMEM((_C,), jnp.int32) for _ in range(4)],
        [pltpu.VMEM((_C, _H), jnp.float32) for _ in range(2)],
        [pltpu.VMEM((_C, _H), jnp.float32) for _ in range(2)],
        pltpu.VMEM_SHARED((_N, _H), jnp.float32),
        [pltpu.SemaphoreType.DMA for _ in range(4)],
        [pltpu.SemaphoreType.DMA for _ in range(2)],
        [pltpu.SemaphoreType.DMA for _ in range(2)],
    ],
)


_BE = 3200


def _edge_mlp_body(ea_ref, w_ref, b_ref, o0_ref, o1_ref, o2_ref):
    y = jnp.dot(ea_ref[...], w_ref[...],
                preferred_element_type=jnp.float32) + b_ref[...]
    o0_ref[...] = y[:, 0:_H]
    o1_ref[...] = y[:, _H:2 * _H]
    o2_ref[...] = y[:, 2 * _H:3 * _H]


def _edge_mlp(ea, wcat, bcat):
    eshape = jax.ShapeDtypeStruct((_E, _H), jnp.float32)
    return pl.pallas_call(
        _edge_mlp_body,
        grid=(_E // _BE,),
        in_specs=[
            pl.BlockSpec((_BE, _D_E), lambda i: (i, 0)),
            pl.BlockSpec((_D_E, 3 * _H), lambda i: (0, 0)),
            pl.BlockSpec((1, 3 * _H), lambda i: (0, 0)),
        ],
        out_specs=[
            pl.BlockSpec((_BE, _H), lambda i: (i, 0)),
            pl.BlockSpec((_BE, _H), lambda i: (i, 0)),
            pl.BlockSpec((_BE, _H), lambda i: (i, 0)),
        ],
        out_shape=[eshape, eshape, eshape],
    )(ea, wcat, bcat)


def _encoder_body(x_ref, ef_ref, wx_ref, we_ref, b_ref, g_ref, bt_ref, o_ref):
    h = jnp.dot(x_ref[...], wx_ref[...], preferred_element_type=jnp.float32)
    h = h + jnp.dot(ef_ref[...], we_ref[...],
                    preferred_element_type=jnp.float32) + b_ref[...]
    h = jnp.maximum(h, 0.0)
    m = jnp.mean(h, axis=0, keepdims=True)
    v = jnp.mean((h - m) ** 2, axis=0, keepdims=True)
    o_ref[...] = g_ref[...] * (h - m) * lax.rsqrt(v + 1e-5) + bt_ref[...]


def _encoder(x, emb_full, wx, wemb, b2, g2, bt2):
    return pl.pallas_call(
        _encoder_body,
        out_shape=jax.ShapeDtypeStruct((_N, _H), jnp.float32),
    )(x, emb_full, wx, wemb, b2, g2, bt2)


def _node_mlp_body(with_bn, h_ref, ag_ref, wa_ref, ba_ref, wb_ref, bb_ref,
                   g_ref, bt_ref, o_ref):
    hs = h_ref[...] + ag_ref[0] + ag_ref[1]
    t = jnp.maximum(
        jnp.dot(hs, wa_ref[...], preferred_element_type=jnp.float32)
        + ba_ref[...], 0.0)
    t = jnp.dot(t, wb_ref[...], preferred_element_type=jnp.float32) + bb_ref[...]
    if with_bn:
        m = jnp.mean(t, axis=0, keepdims=True)
        v = jnp.mean((t - m) ** 2, axis=0, keepdims=True)
        t = g_ref[...] * (t - m) * lax.rsqrt(v + 1e-5) + bt_ref[...]
        t = jnp.maximum(t, 0.0)
    else:
        t = jnp.maximum(t, 0.0)
    o_ref[...] = t


def _node_mlp(with_bn, h, agg, wa, ba2, wb, bb2, g2, bt2):
    return pl.pallas_call(
        functools.partial(_node_mlp_body, with_bn),
        out_shape=jax.ShapeDtypeStruct((_N, _H), jnp.float32),
    )(h, agg, wa, ba2, wb, bb2, g2, bt2)


def _readout_body(h_ref, batch_ref, wl1_ref, bl1_ref, wl2_ref, bl2_ref, o_ref):
    gids = lax.broadcasted_iota(jnp.int32, (_N, _NGRAPH), 1)
    onehot = (batch_ref[...] == gids).astype(jnp.float32)
    pooled = lax.dot_general(onehot, h_ref[...], (((0,), (0,)), ((), ())),
                             preferred_element_type=jnp.float32)
    t = jnp.dot(jnp.maximum(pooled, 0.0), wl1_ref[...],
                preferred_element_type=jnp.float32) + bl1_ref[...]
    o_ref[...] = jnp.dot(t, wl2_ref[...],
                         preferred_element_type=jnp.float32) + bl2_ref[...]


def _readout(h, batch2d, wl1, bl1_2, wl2, bl2_2):
    return pl.pallas_call(
        _readout_body,
        out_shape=jax.ShapeDtypeStruct((_NGRAPH, _OUT), jnp.float32),
    )(h, batch2d, wl1, bl1_2, wl2, bl2_2)


def kernel(x, edge_index, edge_attr, batch, emb, W_enc, b_enc, g0, bt0,
           We0, be0, W0a, b0a, W0b, b0b,
           We1, be1, W1a, b1a, W1b, b1b,
           We2, be2, W2a, b2a, W2b, b2b,
           g_bn, b_bn, Wl1, bl1, Wl2, bl2):
    row2 = lambda v: v.reshape(1, -1)
    src = edge_index[0]
    dst = edge_index[1]

    wcat = jnp.concatenate([We0, We1, We2], axis=1)
    bcat = jnp.concatenate([be0, be1, be2]).reshape(1, 3 * _H)
    e0, e1, e2 = _edge_mlp(edge_attr, wcat, bcat)

    emb_full = jnp.concatenate(
        [emb, jnp.broadcast_to(emb[0:1], (_N - _NG, _EMB))], axis=0)
    h = _encoder(x, emb_full, W_enc[:_D_IN], W_enc[_D_IN:],
                 row2(b_enc), row2(g0), row2(bt0))

    agg = _sc_gine(h, src, dst, e0)
    h = _node_mlp(False, h, agg, W0a, row2(b0a), W0b, row2(b0b),
                  row2(g_bn), row2(b_bn))
    agg = _sc_gine(h, src, dst, e1)
    h = _node_mlp(True, h, agg, W1a, row2(b1a), W1b, row2(b1b),
                  row2(g_bn), row2(b_bn))
    agg = _sc_gine(h, src, dst, e2)
    h = _node_mlp(True, h, agg, W2a, row2(b2a), W2b, row2(b2b),
                  row2(g_bn), row2(b_bn))

    return _readout(h, batch.reshape(_N, 1), Wl1, row2(bl1), Wl2, row2(bl2))


# speedup vs baseline: 4.6495x; 1.7537x over previous
import functools

import jax
import jax.numpy as jnp
from jax import lax
from jax.experimental import pallas as pl
from jax.experimental.pallas import tpu as pltpu
from jax.experimental.pallas import tpu_sc as plsc

_N = 10000
_E = 320000
_D_IN = 128
_H = 128
_D_E = 16
_NG = 8
_EMB = 16
_NGRAPH = 64
_OUT = 8

_NC = 2
_NS = 16
_L = 16
_NW = _NC * _NS
_EPT = _E // _NW
_C = 40
_NCHUNK = _EPT // _C
_RC = 40
_NRCHUNK = _N // _RC


def _sc_gine_body(h_hbm, src_hbm, dst_hbm, emat_hbm, out_hbm,
                  svs, dvs, hrs, ers, msg, agg_sh, sis, sgs, ses):
    cid = lax.axis_index("c")
    sid = lax.axis_index("s")
    wid = sid * _NC + cid
    nvec = _H // _L
    ntrip = (_NRCHUNK - sid + _NS - 1) // _NS

    zero = jnp.zeros((_L,), jnp.float32)

    @pl.loop(0, _C)
    def _(i):
        for j in range(nvec):
            ers[0][i, pl.ds(j * _L, _L)] = zero

    @pl.loop(0, ntrip)
    def _(k):
        pltpu.sync_copy(ers[0], agg_sh.at[pl.ds((sid + k * _NS) * _RC, _RC)])

    plsc.subcore_barrier()

    ebase = wid * _EPT

    def issue_idx(c, q):
        b = ebase + c * _C
        pltpu.async_copy(src_hbm.at[pl.ds(b, _C)], svs[q], sis[q])
        pltpu.async_copy(dst_hbm.at[pl.ds(b, _C)], dvs[q], sis[q])

    def wait_idx(c, q):
        b = ebase + c * _C
        pltpu.make_async_copy(src_hbm.at[pl.ds(b, _C)], svs[q], sis[q]).wait()
        pltpu.make_async_copy(dst_hbm.at[pl.ds(b, _C)], dvs[q], sis[q]).wait()

    def issue_rows(c, q, b):
        pltpu.async_copy(h_hbm.at[svs[q]], hrs[b], sgs[b])
        pltpu.async_copy(emat_hbm.at[pl.ds(ebase + c * _C, _C)], ers[b],
                         ses[b])

    def wait_rows(c, q, b):
        pltpu.make_async_copy(h_hbm.at[svs[q]], hrs[b], sgs[b]).wait()
        pltpu.make_async_copy(emat_hbm.at[pl.ds(ebase + c * _C, _C)], ers[b],
                              ses[b]).wait()

    def compute(b):
        hr = hrs[b]
        er = ers[b]

        @plsc.parallel_loop(0, _C, unroll=8)
        def _(i):
            for j in range(nvec):
                sl = pl.ds(j * _L, _L)
                msg[i, sl] = jnp.maximum(hr[i, sl] + er[i, sl], 0.0)

    def step(c, q, b, steady):
        wait_rows(c, q, b)
        compute(b)
        pltpu.sync_copy(msg, agg_sh.at[dvs[q]], add=True)
        if steady:
            @pl.when(c + 4 < _NCHUNK)
            def _():
                issue_idx(c + 4, q)

            @pl.when(c + 2 < _NCHUNK)
            def _():
                wait_idx(c + 2, (q + 2) % 4)
                issue_rows(c + 2, (q + 2) % 4, b)

    for q in range(4):
        issue_idx(q, q)
    for q in range(2):
        wait_idx(q, q)
        issue_rows(q, q, q)

    @pl.loop(0, _NCHUNK // 4)
    def _(m):
        c = 4 * m
        for u in range(4):
            step(c + u, u, u % 2, True)

    for u in range(_NCHUNK % 4):
        step((_NCHUNK // 4) * 4 + u, u, u % 2, False)
    plsc.subcore_barrier()

    @pl.loop(0, ntrip)
    def _(k):
        r = (sid + k * _NS) * _RC
        pltpu.sync_copy(agg_sh.at[pl.ds(r, _RC)], ers[0])
        pltpu.sync_copy(ers[0], out_hbm.at[cid, pl.ds(r, _RC)])


_sc_gine = pl.kernel(
    _sc_gine_body,
    out_type=jax.ShapeDtypeStruct((_NC, _N, _H), jnp.float32),
    mesh=plsc.VectorSubcoreMesh(core_axis_name="c", subcore_axis_name="s"),
    scratch_types=[
        [pltpu.VMEM((_C,), jnp.int32) for _ in range(4)],
        [pltpu.VMEM((_C,), jnp.int32) for _ in range(4)],
        [pltpu.VMEM((_C, _H), jnp.float32) for _ in range(2)],
        [pltpu.VMEM((_C, _H), jnp.float32) for _ in range(2)],
        pltpu.VMEM((_C, _H), jnp.float32),
        pltpu.VMEM_SHARED((_N, _H), jnp.float32),
        [pltpu.SemaphoreType.DMA for _ in range(4)],
        [pltpu.SemaphoreType.DMA for _ in range(2)],
        [pltpu.SemaphoreType.DMA for _ in range(2)],
    ],
)


_BE = 3200


def _edge_mlp_body(ea_ref, w_ref, b_ref, o0_ref, o1_ref, o2_ref):
    y = jnp.dot(ea_ref[...], w_ref[...],
                preferred_element_type=jnp.float32) + b_ref[...]
    o0_ref[...] = y[:, 0:_H]
    o1_ref[...] = y[:, _H:2 * _H]
    o2_ref[...] = y[:, 2 * _H:3 * _H]


def _edge_mlp(ea, wcat, bcat):
    eshape = jax.ShapeDtypeStruct((_E, _H), jnp.float32)
    return pl.pallas_call(
        _edge_mlp_body,
        grid=(_E // _BE,),
        in_specs=[
            pl.BlockSpec((_BE, _D_E), lambda i: (i, 0)),
            pl.BlockSpec((_D_E, 3 * _H), lambda i: (0, 0)),
            pl.BlockSpec((1, 3 * _H), lambda i: (0, 0)),
        ],
        out_specs=[
            pl.BlockSpec((_BE, _H), lambda i: (i, 0)),
            pl.BlockSpec((_BE, _H), lambda i: (i, 0)),
            pl.BlockSpec((_BE, _H), lambda i: (i, 0)),
        ],
        out_shape=[eshape, eshape, eshape],
    )(ea, wcat, bcat)


def _encoder_body(x_ref, ef_ref, wx_ref, we_ref, b_ref, g_ref, bt_ref, o_ref):
    h = jnp.dot(x_ref[...], wx_ref[...], preferred_element_type=jnp.float32)
    h = h + jnp.dot(ef_ref[...], we_ref[...],
                    preferred_element_type=jnp.float32) + b_ref[...]
    h = jnp.maximum(h, 0.0)
    m = jnp.mean(h, axis=0, keepdims=True)
    v = jnp.mean((h - m) ** 2, axis=0, keepdims=True)
    o_ref[...] = g_ref[...] * (h - m) * lax.rsqrt(v + 1e-5) + bt_ref[...]


def _encoder(x, emb_full, wx, wemb, b2, g2, bt2):
    return pl.pallas_call(
        _encoder_body,
        out_shape=jax.ShapeDtypeStruct((_N, _H), jnp.float32),
    )(x, emb_full, wx, wemb, b2, g2, bt2)


def _node_mlp_body(with_bn, h_ref, ag_ref, wa_ref, ba_ref, wb_ref, bb_ref,
                   g_ref, bt_ref, o_ref):
    hs = h_ref[...] + ag_ref[0] + ag_ref[1]
    t = jnp.maximum(
        jnp.dot(hs, wa_ref[...], preferred_element_type=jnp.float32)
        + ba_ref[...], 0.0)
    t = jnp.dot(t, wb_ref[...], preferred_element_type=jnp.float32) + bb_ref[...]
    if with_bn:
        m = jnp.mean(t, axis=0, keepdims=True)
        v = jnp.mean((t - m) ** 2, axis=0, keepdims=True)
        t = g_ref[...] * (t - m) * lax.rsqrt(v + 1e-5) + bt_ref[...]
        t = jnp.maximum(t, 0.0)
    else:
        t = jnp.maximum(t, 0.0)
    o_ref[...] = t


def _node_mlp(with_bn, h, agg, wa, ba2, wb, bb2, g2, bt2):
    return pl.pallas_call(
        functools.partial(_node_mlp_body, with_bn),
        out_shape=jax.ShapeDtypeStruct((_N, _H), jnp.float32),
    )(h, agg, wa, ba2, wb, bb2, g2, bt2)


def _readout_body(h_ref, batch_ref, wl1_ref, bl1_ref, wl2_ref, bl2_ref, o_ref):
    gids = lax.broadcasted_iota(jnp.int32, (_N, _NGRAPH), 1)
    onehot = (batch_ref[...] == gids).astype(jnp.float32)
    pooled = lax.dot_general(onehot, h_ref[...], (((0,), (0,)), ((), ())),
                             preferred_element_type=jnp.float32)
    t = jnp.dot(jnp.maximum(pooled, 0.0), wl1_ref[...],
                preferred_element_type=jnp.float32) + bl1_ref[...]
    o_ref[...] = jnp.dot(t, wl2_ref[...],
                         preferred_element_type=jnp.float32) + bl2_ref[...]


def _readout(h, batch2d, wl1, bl1_2, wl2, bl2_2):
    return pl.pallas_call(
        _readout_body,
        out_shape=jax.ShapeDtypeStruct((_NGRAPH, _OUT), jnp.float32),
    )(h, batch2d, wl1, bl1_2, wl2, bl2_2)


def kernel(x, edge_index, edge_attr, batch, emb, W_enc, b_enc, g0, bt0,
           We0, be0, W0a, b0a, W0b, b0b,
           We1, be1, W1a, b1a, W1b, b1b,
           We2, be2, W2a, b2a, W2b, b2b,
           g_bn, b_bn, Wl1, bl1, Wl2, bl2):
    row2 = lambda v: v.reshape(1, -1)
    src = edge_index[0]
    dst = edge_index[1]

    wcat = jnp.concatenate([We0, We1, We2], axis=1)
    bcat = jnp.concatenate([be0, be1, be2]).reshape(1, 3 * _H)
    e0, e1, e2 = _edge_mlp(edge_attr, wcat, bcat)

    emb_full = jnp.concatenate(
        [emb, jnp.broadcast_to(emb[0:1], (_N - _NG, _EMB))], axis=0)
    h = _encoder(x, emb_full, W_enc[:_D_IN], W_enc[_D_IN:],
                 row2(b_enc), row2(g0), row2(bt0))

    agg = _sc_gine(h, src, dst, e0)
    h = _node_mlp(False, h, agg, W0a, row2(b0a), W0b, row2(b0b),
                  row2(g_bn), row2(b_bn))
    agg = _sc_gine(h, src, dst, e1)
    h = _node_mlp(True, h, agg, W1a, row2(b1a), W1b, row2(b1b),
                  row2(g_bn), row2(b_bn))
    agg = _sc_gine(h, src, dst, e2)
    h = _node_mlp(True, h, agg, W2a, row2(b2a), W2b, row2(b2b),
                  row2(g_bn), row2(b_bn))

    return _readout(h, batch.reshape(_N, 1), Wl1, row2(bl1), Wl2, row2(bl2))


# speedup vs baseline: 5.3833x; 1.1578x over previous
import functools

import jax
import jax.numpy as jnp
from jax import lax
from jax.experimental import pallas as pl
from jax.experimental.pallas import tpu as pltpu
from jax.experimental.pallas import tpu_sc as plsc

_N = 10000
_E = 320000
_D_IN = 128
_H = 128
_D_E = 16
_NG = 8
_EMB = 16
_NGRAPH = 64
_OUT = 8

_NC = 2
_NS = 16
_L = 16
_NW = _NC * _NS
_EPT = _E // _NW
_C = 80
_NCHUNK = _EPT // _C
_RC = 80
_NRCHUNK = _N // _RC


def _sc_gine_body(h_hbm, src_hbm, dst_hbm, emat_hbm, out_hbm,
                  svs, dvs, hrs, er, msg, agg_sh, sis, sgs, se):
    cid = lax.axis_index("c")
    sid = lax.axis_index("s")
    wid = sid * _NC + cid
    nvec = _H // _L
    ntrip = (_NRCHUNK - sid + _NS - 1) // _NS

    zero = jnp.zeros((_L,), jnp.float32)

    @pl.loop(0, _C)
    def _(i):
        for j in range(nvec):
            msg[i, pl.ds(j * _L, _L)] = zero

    @pl.loop(0, ntrip)
    def _(k):
        pltpu.sync_copy(msg, agg_sh.at[pl.ds((sid + k * _NS) * _RC, _RC)])

    plsc.subcore_barrier()

    ebase = wid * _EPT

    def issue_idx(c, q):
        b = ebase + c * _C
        pltpu.async_copy(src_hbm.at[pl.ds(b, _C)], svs[q], sis[q])
        pltpu.async_copy(dst_hbm.at[pl.ds(b, _C)], dvs[q], sis[q])

    def wait_idx(c, q):
        b = ebase + c * _C
        pltpu.make_async_copy(src_hbm.at[pl.ds(b, _C)], svs[q], sis[q]).wait()
        pltpu.make_async_copy(dst_hbm.at[pl.ds(b, _C)], dvs[q], sis[q]).wait()

    def issue_gather(q, b):
        pltpu.async_copy(h_hbm.at[svs[q]], hrs[b], sgs[b])

    def wait_gather(q, b):
        pltpu.make_async_copy(h_hbm.at[svs[q]], hrs[b], sgs[b]).wait()

    def issue_e(c):
        pltpu.async_copy(emat_hbm.at[pl.ds(ebase + c * _C, _C)], er, se)

    def wait_e(c):
        pltpu.make_async_copy(emat_hbm.at[pl.ds(ebase + c * _C, _C)], er,
                              se).wait()

    def compute(b):
        hr = hrs[b]

        @plsc.parallel_loop(0, _C, unroll=8)
        def _(i):
            for g in range(nvec // 2):
                ei = er[i, pl.ds(g * _L, _L)]
                ea = lax.bitcast_convert_type(ei << 16, jnp.float32)
                eb = lax.bitcast_convert_type(ei & jnp.int32(-65536), jnp.float32)
                sl0 = pl.ds((2 * g) * _L, _L)
                sl1 = pl.ds((2 * g + 1) * _L, _L)
                msg[i, sl0] = jnp.maximum(hr[i, sl0] + ea, 0.0)
                msg[i, sl1] = jnp.maximum(hr[i, sl1] + eb, 0.0)

    def step(c, q, b):
        wait_gather(q, b)
        wait_e(c)
        compute(b)

        @pl.when(c + 1 < _NCHUNK)
        def _():
            issue_e(c + 1)

        pltpu.sync_copy(msg, agg_sh.at[dvs[q]], add=True)

        @pl.when(c + 4 < _NCHUNK)
        def _():
            issue_idx(c + 4, q)

        @pl.when(c + 2 < _NCHUNK)
        def _():
            wait_idx(c + 2, (q + 2) % 4)
            issue_gather((q + 2) % 4, b)

    for q in range(4):
        issue_idx(q, q)
    issue_e(0)
    for q in range(2):
        wait_idx(q, q)
        issue_gather(q, q)

    @pl.loop(0, _NCHUNK // 4)
    def _(m):
        c = 4 * m
        for u in range(4):
            step(c + u, u, u % 2)

    for u in range(_NCHUNK % 4):
        step((_NCHUNK // 4) * 4 + u, u, u % 2)
    plsc.subcore_barrier()

    @pl.loop(0, ntrip)
    def _(k):
        r = (sid + k * _NS) * _RC
        pltpu.sync_copy(agg_sh.at[pl.ds(r, _RC)], msg)
        pltpu.sync_copy(msg, out_hbm.at[cid, pl.ds(r, _RC)])


_sc_gine = pl.kernel(
    _sc_gine_body,
    out_type=jax.ShapeDtypeStruct((_NC, _N, _H), jnp.float32),
    mesh=plsc.VectorSubcoreMesh(core_axis_name="c", subcore_axis_name="s"),
    scratch_types=[
        [pltpu.VMEM((_C,), jnp.int32) for _ in range(4)],
        [pltpu.VMEM((_C,), jnp.int32) for _ in range(4)],
        [pltpu.VMEM((_C, _H), jnp.float32) for _ in range(2)],
        pltpu.VMEM((_C, _H // 2), jnp.int32),
        pltpu.VMEM((_C, _H), jnp.float32),
        pltpu.VMEM_SHARED((_N, _H), jnp.float32),
        [pltpu.SemaphoreType.DMA for _ in range(4)],
        [pltpu.SemaphoreType.DMA for _ in range(2)],
        pltpu.SemaphoreType.DMA,
    ],
)


_BE = 3200


def _bf16_bits(y):
    u = lax.bitcast_convert_type(y, jnp.uint32)
    return u + jnp.uint32(0x7FFF) + ((u >> 16) & jnp.uint32(1))


def _edge_mlp_body(eat_ref, wa_ref, ba_ref, wb_ref, bb_ref, o_ref):
    ya = lax.dot_general(
        eat_ref[...], wa_ref[...], (((0,), (0,)), ((), ())),
        preferred_element_type=jnp.float32) + ba_ref[...]
    yb = lax.dot_general(
        eat_ref[...], wb_ref[...], (((0,), (0,)), ((), ())),
        preferred_element_type=jnp.float32) + bb_ref[...]
    packed = ((_bf16_bits(ya) >> 16)
              | (_bf16_bits(yb) & jnp.uint32(0xFFFF0000)))
    o_ref[...] = lax.bitcast_convert_type(packed, jnp.int32)


def _edge_mlp(ea_t, wa, ba, wb, bb):
    return pl.pallas_call(
        _edge_mlp_body,
        grid=(_E // _BE,),
        in_specs=[
            pl.BlockSpec((_D_E, _BE), lambda i: (0, i)),
            pl.BlockSpec((_D_E, _H // 2), lambda i: (0, 0)),
            pl.BlockSpec((1, _H // 2), lambda i: (0, 0)),
            pl.BlockSpec((_D_E, _H // 2), lambda i: (0, 0)),
            pl.BlockSpec((1, _H // 2), lambda i: (0, 0)),
        ],
        out_specs=pl.BlockSpec((_BE, _H // 2), lambda i: (i, 0)),
        out_shape=jax.ShapeDtypeStruct((_E, _H // 2), jnp.int32),
    )(ea_t, wa, ba, wb, bb)


def _encoder_body(x_ref, ef_ref, wx_ref, we_ref, b_ref, g_ref, bt_ref, o_ref):
    h = jnp.dot(x_ref[...], wx_ref[...], preferred_element_type=jnp.float32)
    h = h + jnp.dot(ef_ref[...], we_ref[...],
                    preferred_element_type=jnp.float32) + b_ref[...]
    h = jnp.maximum(h, 0.0)
    m = jnp.mean(h, axis=0, keepdims=True)
    v = jnp.mean((h - m) ** 2, axis=0, keepdims=True)
    o_ref[...] = g_ref[...] * (h - m) * lax.rsqrt(v + 1e-5) + bt_ref[...]


def _encoder(x, emb_full, wx, wemb, b2, g2, bt2):
    return pl.pallas_call(
        _encoder_body,
        out_shape=jax.ShapeDtypeStruct((_N, _H), jnp.float32),
    )(x, emb_full, wx, wemb, b2, g2, bt2)


def _node_mlp_body(with_bn, h_ref, ag_ref, wa_ref, ba_ref, wb_ref, bb_ref,
                   g_ref, bt_ref, o_ref):
    hs = h_ref[...] + ag_ref[0] + ag_ref[1]
    t = jnp.maximum(
        jnp.dot(hs, wa_ref[...], preferred_element_type=jnp.float32)
        + ba_ref[...], 0.0)
    t = jnp.dot(t, wb_ref[...], preferred_element_type=jnp.float32) + bb_ref[...]
    if with_bn:
        m = jnp.mean(t, axis=0, keepdims=True)
        v = jnp.mean((t - m) ** 2, axis=0, keepdims=True)
        t = g_ref[...] * (t - m) * lax.rsqrt(v + 1e-5) + bt_ref[...]
        t = jnp.maximum(t, 0.0)
    else:
        t = jnp.maximum(t, 0.0)
    o_ref[...] = t


def _node_mlp(with_bn, h, agg, wa, ba2, wb, bb2, g2, bt2):
    return pl.pallas_call(
        functools.partial(_node_mlp_body, with_bn),
        out_shape=jax.ShapeDtypeStruct((_N, _H), jnp.float32),
    )(h, agg, wa, ba2, wb, bb2, g2, bt2)


def _readout_body(h_ref, batch_ref, wl1_ref, bl1_ref, wl2_ref, bl2_ref, o_ref):
    gids = lax.broadcasted_iota(jnp.int32, (_N, _NGRAPH), 1)
    onehot = (batch_ref[...] == gids).astype(jnp.float32)
    pooled = lax.dot_general(onehot, h_ref[...], (((0,), (0,)), ((), ())),
                             preferred_element_type=jnp.float32)
    t = jnp.dot(jnp.maximum(pooled, 0.0), wl1_ref[...],
                preferred_element_type=jnp.float32) + bl1_ref[...]
    o_ref[...] = jnp.dot(t, wl2_ref[...],
                         preferred_element_type=jnp.float32) + bl2_ref[...]


def _readout(h, batch2d, wl1, bl1_2, wl2, bl2_2):
    return pl.pallas_call(
        _readout_body,
        out_shape=jax.ShapeDtypeStruct((_NGRAPH, _OUT), jnp.float32),
    )(h, batch2d, wl1, bl1_2, wl2, bl2_2)


def kernel(x, edge_index, edge_attr, batch, emb, W_enc, b_enc, g0, bt0,
           We0, be0, W0a, b0a, W0b, b0b,
           We1, be1, W1a, b1a, W1b, b1b,
           We2, be2, W2a, b2a, W2b, b2b,
           g_bn, b_bn, Wl1, bl1, Wl2, bl2):
    row2 = lambda v: v.reshape(1, -1)
    src = edge_index[0]
    dst = edge_index[1]

    ea_t = edge_attr.T
    pa = jnp.array([32 * (j // 16) + j % 16 for j in range(64)], jnp.int32)
    pb = pa + 16
    e0 = _edge_mlp(ea_t, We0[:, pa], row2(be0[pa]), We0[:, pb], row2(be0[pb]))
    e1 = _edge_mlp(ea_t, We1[:, pa], row2(be1[pa]), We1[:, pb], row2(be1[pb]))
    e2 = _edge_mlp(ea_t, We2[:, pa], row2(be2[pa]), We2[:, pb], row2(be2[pb]))

    emb_full = jnp.concatenate(
        [emb, jnp.broadcast_to(emb[0:1], (_N - _NG, _EMB))], axis=0)
    h = _encoder(x, emb_full, W_enc[:_D_IN], W_enc[_D_IN:],
                 row2(b_enc), row2(g0), row2(bt0))

    agg = _sc_gine(h, src, dst, e0)
    h = _node_mlp(False, h, agg, W0a, row2(b0a), W0b, row2(b0b),
                  row2(g_bn), row2(b_bn))
    agg = _sc_gine(h, src, dst, e1)
    h = _node_mlp(True, h, agg, W1a, row2(b1a), W1b, row2(b1b),
                  row2(g_bn), row2(b_bn))
    agg = _sc_gine(h, src, dst, e2)
    h = _node_mlp(True, h, agg, W2a, row2(b2a), W2b, row2(b2b),
                  row2(g_bn), row2(b_bn))

    return _readout(h, batch.reshape(_N, 1), Wl1, row2(bl1), Wl2, row2(bl2))


# speedup vs baseline: 5.6352x; 1.0468x over previous
import functools

import jax
import jax.numpy as jnp
from jax import lax
from jax.experimental import pallas as pl
from jax.experimental.pallas import tpu as pltpu
from jax.experimental.pallas import tpu_sc as plsc

_N = 10000
_E = 320000
_D_IN = 128
_H = 128
_D_E = 16
_NG = 8
_EMB = 16
_NGRAPH = 64
_OUT = 8

_NC = 2
_NS = 16
_L = 16
_NW = _NC * _NS
_EPT = _E // _NW
_C = 40
_NCHUNK = _EPT // _C
_RC = 40
_NRCHUNK = _N // _RC


def _sc_gine_body(h_hbm, src_hbm, dst_hbm, emat_hbm, out_hbm,
                  svs, dvs, hrs, ers, msg, agg_sh, sis, sgs, ses):
    cid = lax.axis_index("c")
    sid = lax.axis_index("s")
    wid = sid * _NC + cid
    nvec = _H // _L
    ntrip = (_NRCHUNK - sid + _NS - 1) // _NS

    zero = jnp.zeros((_L,), jnp.float32)

    @pl.loop(0, _C)
    def _(i):
        for j in range(nvec):
            msg[i, pl.ds(j * _L, _L)] = zero

    @pl.loop(0, ntrip)
    def _(k):
        pltpu.sync_copy(msg, agg_sh.at[pl.ds((sid + k * _NS) * _RC, _RC)])

    plsc.subcore_barrier()

    ebase = wid * _EPT

    def issue_idx(c, q):
        b = ebase + c * _C
        pltpu.async_copy(src_hbm.at[pl.ds(b, _C)], svs[q], sis[q])
        pltpu.async_copy(dst_hbm.at[pl.ds(b, _C)], dvs[q], sis[q])

    def wait_idx(c, q):
        b = ebase + c * _C
        pltpu.make_async_copy(src_hbm.at[pl.ds(b, _C)], svs[q], sis[q]).wait()
        pltpu.make_async_copy(dst_hbm.at[pl.ds(b, _C)], dvs[q], sis[q]).wait()

    def issue_rows(c, q, b):
        pltpu.async_copy(h_hbm.at[svs[q]], hrs[b], sgs[b])
        pltpu.async_copy(emat_hbm.at[pl.ds(ebase + c * _C, _C)], ers[b],
                         ses[b])

    def wait_rows(c, q, b):
        pltpu.make_async_copy(h_hbm.at[svs[q]], hrs[b], sgs[b]).wait()
        pltpu.make_async_copy(emat_hbm.at[pl.ds(ebase + c * _C, _C)], ers[b],
                              ses[b]).wait()

    def compute(b):
        hr = hrs[b]
        er = ers[b]

        @plsc.parallel_loop(0, _C, unroll=8)
        def _(i):
            for g in range(nvec // 2):
                ei = er[i, pl.ds(g * _L, _L)]
                ea = lax.bitcast_convert_type(ei << 16, jnp.float32)
                eb = lax.bitcast_convert_type(ei & jnp.int32(-65536), jnp.float32)
                sl0 = pl.ds((2 * g) * _L, _L)
                sl1 = pl.ds((2 * g + 1) * _L, _L)
                msg[i, sl0] = jnp.maximum(hr[i, sl0] + ea, 0.0)
                msg[i, sl1] = jnp.maximum(hr[i, sl1] + eb, 0.0)

    def step(c, q, b, steady):
        wait_rows(c, q, b)
        compute(b)
        pltpu.sync_copy(msg, agg_sh.at[dvs[q]], add=True)
        if steady:
            @pl.when(c + 4 < _NCHUNK)
            def _():
                issue_idx(c + 4, q)

            @pl.when(c + 2 < _NCHUNK)
            def _():
                wait_idx(c + 2, (q + 2) % 4)
                issue_rows(c + 2, (q + 2) % 4, b)

    for q in range(4):
        issue_idx(q, q)
    for q in range(2):
        wait_idx(q, q)
        issue_rows(q, q, q)

    @pl.loop(0, _NCHUNK // 4)
    def _(m):
        c = 4 * m
        for u in range(4):
            step(c + u, u, u % 2, True)

    for u in range(_NCHUNK % 4):
        step((_NCHUNK // 4) * 4 + u, u, u % 2, False)
    plsc.subcore_barrier()

    @pl.loop(0, ntrip)
    def _(k):
        r = (sid + k * _NS) * _RC
        pltpu.sync_copy(agg_sh.at[pl.ds(r, _RC)], msg)
        pltpu.sync_copy(msg, out_hbm.at[cid, pl.ds(r, _RC)])


_sc_gine = pl.kernel(
    _sc_gine_body,
    out_type=jax.ShapeDtypeStruct((_NC, _N, _H), jnp.float32),
    mesh=plsc.VectorSubcoreMesh(core_axis_name="c", subcore_axis_name="s"),
    scratch_types=[
        [pltpu.VMEM((_C,), jnp.int32) for _ in range(4)],
        [pltpu.VMEM((_C,), jnp.int32) for _ in range(4)],
        [pltpu.VMEM((_C, _H), jnp.float32) for _ in range(2)],
        [pltpu.VMEM((_C, _H // 2), jnp.int32) for _ in range(2)],
        pltpu.VMEM((_C, _H), jnp.float32),
        pltpu.VMEM_SHARED((_N, _H), jnp.float32),
        [pltpu.SemaphoreType.DMA for _ in range(4)],
        [pltpu.SemaphoreType.DMA for _ in range(2)],
        [pltpu.SemaphoreType.DMA for _ in range(2)],
    ],
)


_BE = 6400


def _bf16_bits(y):
    u = lax.bitcast_convert_type(y, jnp.uint32)
    return u + jnp.uint32(0x7FFF) + ((u >> 16) & jnp.uint32(1))


def _edge_mlp_body(eat_ref, w_ref, b_ref, o_ref):
    y = lax.dot_general(
        eat_ref[...], w_ref[...], (((0,), (0,)), ((), ())),
        preferred_element_type=jnp.float32) + b_ref[...]
    packed = ((_bf16_bits(y[:, :_H // 2]) >> 16)
              | (_bf16_bits(y[:, _H // 2:]) & jnp.uint32(0xFFFF0000)))
    o_ref[...] = lax.bitcast_convert_type(packed, jnp.int32)


def _edge_mlp(ea_t, w, b):
    return pl.pallas_call(
        _edge_mlp_body,
        grid=(_E // _BE,),
        in_specs=[
            pl.BlockSpec((_D_E, _BE), lambda i: (0, i)),
            pl.BlockSpec((_D_E, _H), lambda i: (0, 0)),
            pl.BlockSpec((1, _H), lambda i: (0, 0)),
        ],
        out_specs=pl.BlockSpec((_BE, _H // 2), lambda i: (i, 0)),
        out_shape=jax.ShapeDtypeStruct((_E, _H // 2), jnp.int32),
    )(ea_t, w, b)


def _encoder_body(x_ref, ef_ref, wx_ref, we_ref, b_ref, g_ref, bt_ref, o_ref):
    h = jnp.dot(x_ref[...], wx_ref[...], preferred_element_type=jnp.float32)
    h = h + jnp.dot(ef_ref[...], we_ref[...],
                    preferred_element_type=jnp.float32) + b_ref[...]
    h = jnp.maximum(h, 0.0)
    m = jnp.mean(h, axis=0, keepdims=True)
    v = jnp.mean((h - m) ** 2, axis=0, keepdims=True)
    o_ref[...] = g_ref[...] * (h - m) * lax.rsqrt(v + 1e-5) + bt_ref[...]


def _encoder(x, emb_full, wx, wemb, b2, g2, bt2):
    return pl.pallas_call(
        _encoder_body,
        out_shape=jax.ShapeDtypeStruct((_N, _H), jnp.float32),
    )(x, emb_full, wx, wemb, b2, g2, bt2)


def _node_mlp_body(with_bn, h_ref, ag_ref, wa_ref, ba_ref, wb_ref, bb_ref,
                   g_ref, bt_ref, o_ref):
    hs = h_ref[...] + ag_ref[0] + ag_ref[1]
    t = jnp.maximum(
        jnp.dot(hs, wa_ref[...], preferred_element_type=jnp.float32)
        + ba_ref[...], 0.0)
    t = jnp.dot(t, wb_ref[...], preferred_element_type=jnp.float32) + bb_ref[...]
    if with_bn:
        m = jnp.mean(t, axis=0, keepdims=True)
        v = jnp.mean((t - m) ** 2, axis=0, keepdims=True)
        t = g_ref[...] * (t - m) * lax.rsqrt(v + 1e-5) + bt_ref[...]
        t = jnp.maximum(t, 0.0)
    else:
        t = jnp.maximum(t, 0.0)
    o_ref[...] = t


def _node_mlp(with_bn, h, agg, wa, ba2, wb, bb2, g2, bt2):
    return pl.pallas_call(
        functools.partial(_node_mlp_body, with_bn),
        out_shape=jax.ShapeDtypeStruct((_N, _H), jnp.float32),
    )(h, agg, wa, ba2, wb, bb2, g2, bt2)


def _readout_body(h_ref, batch_ref, wl1_ref, bl1_ref, wl2_ref, bl2_ref, o_ref):
    gids = lax.broadcasted_iota(jnp.int32, (_N, _NGRAPH), 1)
    onehot = (batch_ref[...] == gids).astype(jnp.float32)
    pooled = lax.dot_general(onehot, h_ref[...], (((0,), (0,)), ((), ())),
                             preferred_element_type=jnp.float32)
    t = jnp.dot(jnp.maximum(pooled, 0.0), wl1_ref[...],
                preferred_element_type=jnp.float32) + bl1_ref[...]
    o_ref[...] = jnp.dot(t, wl2_ref[...],
                         preferred_element_type=jnp.float32) + bl2_ref[...]


def _readout(h, batch2d, wl1, bl1_2, wl2, bl2_2):
    return pl.pallas_call(
        _readout_body,
        out_shape=jax.ShapeDtypeStruct((_NGRAPH, _OUT), jnp.float32),
    )(h, batch2d, wl1, bl1_2, wl2, bl2_2)


def kernel(x, edge_index, edge_attr, batch, emb, W_enc, b_enc, g0, bt0,
           We0, be0, W0a, b0a, W0b, b0b,
           We1, be1, W1a, b1a, W1b, b1b,
           We2, be2, W2a, b2a, W2b, b2b,
           g_bn, b_bn, Wl1, bl1, Wl2, bl2):
    row2 = lambda v: v.reshape(1, -1)
    src = edge_index[0]
    dst = edge_index[1]

    ea_t = edge_attr.T
    pa = jnp.array([32 * (j // 16) + j % 16 for j in range(64)], jnp.int32)
    pcat = jnp.concatenate([pa, pa + 16])
    e0 = _edge_mlp(ea_t, We0[:, pcat], row2(be0[pcat]))
    e1 = _edge_mlp(ea_t, We1[:, pcat], row2(be1[pcat]))
    e2 = _edge_mlp(ea_t, We2[:, pcat], row2(be2[pcat]))

    emb_full = jnp.concatenate(
        [emb, jnp.broadcast_to(emb[0:1], (_N - _NG, _EMB))], axis=0)
    h = _encoder(x, emb_full, W_enc[:_D_IN], W_enc[_D_IN:],
                 row2(b_enc), row2(g0), row2(bt0))

    agg = _sc_gine(h, src, dst, e0)
    h = _node_mlp(False, h, agg, W0a, row2(b0a), W0b, row2(b0b),
                  row2(g_bn), row2(b_bn))
    agg = _sc_gine(h, src, dst, e1)
    h = _node_mlp(True, h, agg, W1a, row2(b1a), W1b, row2(b1b),
                  row2(g_bn), row2(b_bn))
    agg = _sc_gine(h, src, dst, e2)
    h = _node_mlp(True, h, agg, W2a, row2(b2a), W2b, row2(b2b),
                  row2(g_bn), row2(b_bn))

    return _readout(h, batch.reshape(_N, 1), Wl1, row2(bl1), Wl2, row2(bl2))
